# baseline (speedup 1.0000x reference)
import functools

import jax
import jax.numpy as jnp
from jax import lax
from jax.experimental import pallas as pl
from jax.experimental.pallas import tpu as pltpu
from jax.experimental.pallas import tpu_sc as plsc

_NUM_CORES = 2
_NUM_SUBCORES = 16
_NUM_WORKERS = _NUM_CORES * _NUM_SUBCORES
_BCH = 16


@functools.partial(jax.jit, static_argnums=(2, 3, 4))
def _sc_embed(idx2d, table, B, H, D, ):
    per_w = B // _NUM_WORKERS
    n_blocks = per_w // _BCH
    DQ, DR = D // 8, 8
    BQ, BR = B // 128, 128
    blocks_per_bq = BR // _BCH
    mesh = plsc.VectorSubcoreMesh(
        core_axis_name="c", subcore_axis_name="s",
        num_cores=_NUM_CORES, num_subcores=_NUM_SUBCORES)

    @functools.partial(
        pl.kernel,
        out_type=jax.ShapeDtypeStruct((H, DQ, BQ, DR, BR), jnp.float32),
        mesh=mesh,
        scratch_types=[
            pltpu.VMEM((_BCH, H), jnp.int32),
            pltpu.VMEM((_BCH, H), jnp.int32),
            pltpu.VMEM((_BCH, H, D), jnp.float32),
            pltpu.VMEM((_BCH, H, D), jnp.float32),
            pltpu.VMEM((H, DQ, 1, DR, _BCH), jnp.float32),
            pltpu.VMEM((H, DQ, 1, DR, _BCH), jnp.float32),
            pltpu.SemaphoreType.DMA,
            pltpu.SemaphoreType.DMA,
            pltpu.SemaphoreType.DMA,
            pltpu.SemaphoreType.DMA,
            pltpu.SemaphoreType.DMA,
            pltpu.SemaphoreType.DMA,
        ],
        compiler_params=pltpu.CompilerParams(
            use_tc_tiling_on_sc=False, needs_layout_passes=False),
    )
    def k(idx_hbm, table_hbm, out_hbm,
          idx0, idx1, rows0, rows1, perm0, perm1,
          si0, si1, sg0, sg1, ss0, ss1):
        wid = lax.axis_index("s") * _NUM_CORES + lax.axis_index("c")
        base = wid * per_w
        idx_v = (idx0, idx1)
        rows_v = (rows0, rows1)
        perm_v = (perm0, perm1)
        si = (si0, si1)
        sg = (sg0, sg1)
        ss = (ss0, ss1)

        lane = jax.lax.broadcasted_iota(jnp.int32, (16,), 0)

        def start_idx(i, s):
            pltpu.async_copy(idx_hbm.at[pl.ds(base + i * _BCH, _BCH)],
                             idx_v[s], si[s])

        def wait_idx(s):
            pltpu.make_async_copy(idx_hbm.at[pl.ds(base, _BCH)],
                                  idx_v[s], si[s]).wait()

        def start_gathers(s):
            for b in range(_BCH):
                pltpu.async_copy(table_hbm.at[idx_v[s].at[b]],
                                 rows_v[s].at[b], sg[s])

        def wait_gathers(s):
            for b in range(_BCH):
                pltpu.make_async_copy(table_hbm.at[idx_v[s].at[b]],
                                      rows_v[s].at[b], sg[s]).wait()

        def transpose(s):
            rows = rows_v[s]
            perm = perm_v[s]

            def h_body(h, carry):
                h_vec = jnp.full((16,), h, jnp.int32)
                for dq in range(DQ):
                    for dr in range(DR):
                        d_vec = jnp.full((16,), dq * DR + dr, jnp.int32)
                        v = plsc.load_gather(rows, [lane, h_vec, d_vec])
                        perm[h, dq, 0, dr, :] = v
                return carry

            lax.fori_loop(0, H, h_body, 0, unroll=False)

        def start_store(i, s):
            bq = (base + i * _BCH) // BR
            br0 = (i % blocks_per_bq) * _BCH
            pltpu.async_copy(
                perm_v[s],
                out_hbm.at[:, :, pl.ds(bq, 1), :, pl.ds(br0, _BCH)],
                ss[s])

        def wait_store(s):
            pltpu.make_async_copy(
                perm_v[s],
                out_hbm.at[:, :, pl.ds(0, 1), :, pl.ds(0, _BCH)],
                ss[s]).wait()

        start_idx(0, 0)
        start_idx(1, 1)
        wait_idx(0)
        start_gathers(0)
        wait_idx(1)
        start_gathers(1)
        for s in range(2):
            wait_gathers(s)
            transpose(s)
            start_store(s, s)
            start_idx(s + 2, s)
            wait_idx(s)
            start_gathers(s)

        def body(p, carry):
            for s in range(2):
                i = 2 * p + s
                wait_gathers(s)
                wait_store(s)
                transpose(s)
                start_store(i, s)
                start_idx(i + 2, s)
                wait_idx(s)
                start_gathers(s)
            return carry

        lax.fori_loop(1, n_blocks // 2 - 1, body, 0, unroll=False)

        for s in range(2):
            i = n_blocks - 2 + s
            wait_gathers(s)
            wait_store(s)
            transpose(s)
            start_store(i, s)
        wait_store(0)
        wait_store(1)

    return k(idx2d, table)


def kernel(inputs, table):
    B, H = inputs.shape
    V, D = table.shape
    idx2d = inputs.astype(jnp.int32)
    out5 = _sc_embed(idx2d, table, B, H, D)
    return out5.transpose(2, 4, 0, 1, 3).reshape(B, H, D)


# speedup vs baseline: 1.0476x; 1.0476x over previous
import functools

import jax
import jax.numpy as jnp
from jax import lax
from jax.experimental import pallas as pl
from jax.experimental.pallas import tpu as pltpu
from jax.experimental.pallas import tpu_sc as plsc

_NUM_CORES = 2
_NUM_SUBCORES = 16
_NUM_WORKERS = _NUM_CORES * _NUM_SUBCORES
_HB = 5
_BT = 128


@functools.partial(jax.jit, static_argnums=(2, 3, 4))
def _sc_embed(idx_t, table, B, H, D):
    per_w = B // _NUM_WORKERS
    bq_per_w = per_w // _BT
    h_units = H // _HB
    n_units = bq_per_w * h_units
    DQ, DR = D // 8, 8
    BQ = B // _BT
    mesh = plsc.VectorSubcoreMesh(
        core_axis_name="c", subcore_axis_name="s",
        num_cores=_NUM_CORES, num_subcores=_NUM_SUBCORES)

    @functools.partial(
        pl.kernel,
        out_type=jax.ShapeDtypeStruct((H, DQ, BQ, DR, _BT), jnp.float32),
        mesh=mesh,
        scratch_types=[
            pltpu.VMEM((_HB, _BT), jnp.int32),
            pltpu.VMEM((_HB, _BT), jnp.int32),
            pltpu.VMEM((_HB * _BT, D), jnp.float32),
            pltpu.VMEM((_HB * _BT, D), jnp.float32),
            pltpu.VMEM((_HB, DQ, 1, DR, _BT), jnp.float32),
            pltpu.VMEM((_HB, DQ, 1, DR, _BT), jnp.float32),
            pltpu.SemaphoreType.DMA,
            pltpu.SemaphoreType.DMA,
            pltpu.SemaphoreType.DMA,
            pltpu.SemaphoreType.DMA,
            pltpu.SemaphoreType.DMA,
            pltpu.SemaphoreType.DMA,
        ],
        compiler_params=pltpu.CompilerParams(
            use_tc_tiling_on_sc=False, needs_layout_passes=False),
    )
    def k(idx_hbm, table_hbm, out_hbm,
          list0, list1, rows0, rows1, perm0, perm1,
          sl0, sl1, sg0, sg1, ss0, ss1):
        wid = lax.axis_index("s") * _NUM_CORES + lax.axis_index("c")
        list_v = (list0, list1)
        rows_v = (rows0, rows1)
        perm_v = (perm0, perm1)
        sl = (sl0, sl1)
        sg = (sg0, sg1)
        ss = (ss0, ss1)

        iota16 = lax.broadcasted_iota(jnp.int32, (16,), 0)
        d_vecs = [jnp.full((16,), d, jnp.int32) for d in range(D)]

        def unit_pos(u):
            return wid * bq_per_w + u // h_units, (u % h_units) * _HB

        def start_list(u, s):
            bq, h0 = unit_pos(u)
            pltpu.async_copy(
                idx_hbm.at[pl.ds(h0, _HB), pl.ds(bq * _BT, _BT)],
                list_v[s], sl[s])

        def wait_list(s):
            pltpu.make_async_copy(
                idx_hbm.at[pl.ds(0, _HB), pl.ds(0, _BT)],
                list_v[s], sl[s]).wait()

        def start_gathers(s):
            for hh in range(_HB):
                pltpu.async_copy(table_hbm.at[list_v[s].at[hh]],
                                 rows_v[s].at[pl.ds(hh * _BT, _BT)], sg[s])

        def wait_gathers(s):
            for hh in range(_HB):
                pltpu.make_async_copy(
                    table_hbm.at[list_v[s].at[hh]],
                    rows_v[s].at[pl.ds(hh * _BT, _BT)], sg[s]).wait()

        def transpose(s):
            rows = rows_v[s]
            perm = perm_v[s]

            def h_body(hh, carry):
                for j in range(_BT // 16):
                    blk = rows.at[pl.ds(hh * _BT + j * 16, 16)]
                    for d in range(D):
                        v = plsc.load_gather(blk, [iota16, d_vecs[d]])
                        perm[hh, d // DR, 0, d % DR, pl.ds(j * 16, 16)] = v
                return carry

            lax.fori_loop(0, _HB, h_body, 0, unroll=False)

        def start_store(u, s):
            bq, h0 = unit_pos(u)
            pltpu.async_copy(
                perm_v[s],
                out_hbm.at[pl.ds(h0, _HB), :, pl.ds(bq, 1), :, :], ss[s])

        def wait_store(s):
            pltpu.make_async_copy(
                perm_v[s],
                out_hbm.at[pl.ds(0, _HB), :, pl.ds(0, 1), :, :], ss[s]).wait()

        start_list(0, 0)
        start_list(1, 1)
        wait_list(0)
        start_gathers(0)
        wait_list(1)
        start_gathers(1)
        for s in range(2):
            wait_gathers(s)
            transpose(s)
            start_store(s, s)
            start_list(s + 2, s)
            wait_list(s)
            start_gathers(s)

        def body(p, carry):
            for s in range(2):
                u = 2 * p + s
                wait_gathers(s)
                wait_store(s)
                transpose(s)
                start_store(u, s)
                start_list(u + 2, s)
                wait_list(s)
                start_gathers(s)
            return carry

        lax.fori_loop(1, n_units // 2 - 1, body, 0, unroll=False)

        for s in range(2):
            u = n_units - 2 + s
            wait_gathers(s)
            wait_store(s)
            transpose(s)
            start_store(u, s)
        wait_store(0)
        wait_store(1)

    return k(idx_t, table)


def kernel(inputs, table):
    B, H = inputs.shape
    V, D = table.shape
    idx_t = inputs.T.astype(jnp.int32)
    out5 = _sc_embed(idx_t, table, B, H, D)
    return out5.transpose(2, 4, 0, 1, 3).reshape(B, H, D)


# speedup vs baseline: 2.0811x; 1.9865x over previous
import functools

import jax
import jax.numpy as jnp
from jax import lax
from jax.experimental import pallas as pl
from jax.experimental.pallas import tpu as pltpu
from jax.experimental.pallas import tpu_sc as plsc

_NUM_CORES = 2
_NUM_SUBCORES = 16
_NUM_WORKERS = _NUM_CORES * _NUM_SUBCORES
_HB = 5
_BT = 128


@functools.partial(jax.jit, static_argnums=(2, 3, 4))
def _sc_embed(idx_t, table, B, H, D):
    per_w = B // _NUM_WORKERS
    bq_per_w = per_w // _BT
    h_units = H // _HB
    n_units = bq_per_w * h_units
    DQ, DR = D // 8, 8
    BQ = B // _BT
    mesh = plsc.VectorSubcoreMesh(
        core_axis_name="c", subcore_axis_name="s",
        num_cores=_NUM_CORES, num_subcores=_NUM_SUBCORES)

    @functools.partial(
        pl.kernel,
        out_type=jax.ShapeDtypeStruct((H, DQ, BQ, DR, _BT), jnp.float32),
        mesh=mesh,
        scratch_types=[
            pltpu.VMEM((_HB, _BT), jnp.int32),
            pltpu.VMEM((_HB, _BT), jnp.int32),
            pltpu.VMEM((_HB * _BT, D), jnp.float32),
            pltpu.VMEM((_HB * _BT, D), jnp.float32),
            pltpu.VMEM((_HB, DQ, 1, DR, _BT), jnp.float32),
            pltpu.VMEM((_HB, DQ, 1, DR, _BT), jnp.float32),
            pltpu.SemaphoreType.DMA,
            pltpu.SemaphoreType.DMA,
            pltpu.SemaphoreType.DMA,
            pltpu.SemaphoreType.DMA,
            pltpu.SemaphoreType.DMA,
            pltpu.SemaphoreType.DMA,
        ],
        compiler_params=pltpu.CompilerParams(
            use_tc_tiling_on_sc=False, needs_layout_passes=False),
    )
    def k(idx_hbm, table_hbm, out_hbm,
          list0, list1, rows0, rows1, perm0, perm1,
          sl0, sl1, sg0, sg1, ss0, ss1):
        wid = lax.axis_index("s") * _NUM_CORES + lax.axis_index("c")
        list_v = (list0, list1)
        rows_v = (rows0, rows1)
        perm_v = (perm0, perm1)
        sl = (sl0, sl1)
        sg = (sg0, sg1)
        ss = (ss0, ss1)

        iota16 = lax.broadcasted_iota(jnp.int32, (16,), 0)
        d_vecs = [jnp.full((16,), d, jnp.int32) for d in range(D)]

        def unit_pos(u):
            return wid * bq_per_w + u // h_units, (u % h_units) * _HB

        def start_list(u, s):
            bq, h0 = unit_pos(u)
            pltpu.async_copy(
                idx_hbm.at[pl.ds(h0, _HB), pl.ds(bq * _BT, _BT)],
                list_v[s], sl[s])

        def wait_list(s):
            pltpu.make_async_copy(
                idx_hbm.at[pl.ds(0, _HB), pl.ds(0, _BT)],
                list_v[s], sl[s]).wait()

        def start_gathers(s):
            for hh in range(_HB):
                pltpu.async_copy(table_hbm.at[list_v[s].at[hh]],
                                 rows_v[s].at[pl.ds(hh * _BT, _BT)], sg[s])

        def wait_gathers(s):
            for hh in range(_HB):
                pltpu.make_async_copy(
                    table_hbm.at[list_v[s].at[hh]],
                    rows_v[s].at[pl.ds(hh * _BT, _BT)], sg[s]).wait()

        def transpose(s):
            rows = rows_v[s]
            perm = perm_v[s]

            @functools.partial(plsc.parallel_loop, 0, _HB)
            def h_body(hh):
                for j in range(_BT // 16):
                    blk = rows.at[pl.ds(hh * _BT + j * 16, 16)]
                    for d in range(D):
                        v = plsc.load_gather(blk, [iota16, d_vecs[d]])
                        perm[hh, d // DR, 0, d % DR, pl.ds(j * 16, 16)] = v

        def start_store(u, s):
            bq, h0 = unit_pos(u)
            pltpu.async_copy(
                perm_v[s],
                out_hbm.at[pl.ds(h0, _HB), :, pl.ds(bq, 1), :, :], ss[s])

        def wait_store(s):
            pltpu.make_async_copy(
                perm_v[s],
                out_hbm.at[pl.ds(0, _HB), :, pl.ds(0, 1), :, :], ss[s]).wait()

        start_list(0, 0)
        start_list(1, 1)
        wait_list(0)
        start_gathers(0)
        wait_list(1)
        start_gathers(1)
        for s in range(2):
            wait_gathers(s)
            transpose(s)
            start_store(s, s)
            start_list(s + 2, s)
            wait_list(s)
            start_gathers(s)

        def body(p, carry):
            for s in range(2):
                u = 2 * p + s
                wait_gathers(s)
                wait_store(s)
                transpose(s)
                start_store(u, s)
                start_list(u + 2, s)
                wait_list(s)
                start_gathers(s)
            return carry

        lax.fori_loop(1, n_units // 2 - 1, body, 0, unroll=False)

        for s in range(2):
            u = n_units - 2 + s
            wait_gathers(s)
            wait_store(s)
            transpose(s)
            start_store(u, s)
        wait_store(0)
        wait_store(1)

    return k(idx_t, table)


def kernel(inputs, table):
    B, H = inputs.shape
    V, D = table.shape
    idx_t = inputs.T.astype(jnp.int32)
    out5 = _sc_embed(idx_t, table, B, H, D)
    return out5.transpose(2, 4, 0, 1, 3).reshape(B, H, D)
